# baseline (speedup 1.0000x reference)
import functools

import jax
import jax.numpy as jnp
from jax import lax
from jax.experimental import pallas as pl
from jax.experimental.pallas import tpu as pltpu
from jax.experimental.pallas import tpu_sc as plsc

_NUM_CORES = 2
_NUM_SUBCORES = 16
_NUM_WORKERS = _NUM_CORES * _NUM_SUBCORES
_L = 16
_W = 128
_UNROLL = 2


@functools.partial(jax.jit, static_argnums=(2, 3))
def _sc_detile(table_t, tail_flat, V, D):
    n_full = V // _W
    rem = V - n_full * _W
    per_w_lo = n_full // _NUM_WORKERS
    n_hi = n_full - per_w_lo * _NUM_WORKERS
    mesh = plsc.VectorSubcoreMesh(core_axis_name="c", subcore_axis_name="s")

    @functools.partial(
        pl.kernel,
        mesh=mesh,
        out_type=jax.ShapeDtypeStruct((V * D,), jnp.float32),
        scratch_types=[
            pltpu.VMEM((D, _W), jnp.float32),
            pltpu.VMEM((D, _W), jnp.float32),
            pltpu.VMEM((_W * D,), jnp.float32),
            pltpu.VMEM((_W * D,), jnp.float32),
            pltpu.VMEM((rem * D,), jnp.float32),
            pltpu.SemaphoreType.DMA,
            pltpu.SemaphoreType.DMA,
            pltpu.SemaphoreType.DMA,
            pltpu.SemaphoreType.DMA,
        ],
        compiler_params=pltpu.CompilerParams(
            use_tc_tiling_on_sc=True, needs_layout_passes=False
        ),
    )
    def k(tt_hbm, tail_hbm, out_hbm, s0, s1, o0, o1, tl, g0, g1, w0, w1):
        slab, obuf = (s0, s1), (o0, o1)
        gsem, wsem = (g0, g1), (w0, w1)
        wid = lax.axis_index("s") * _NUM_CORES + lax.axis_index("c")
        nblk = jnp.where(wid < n_hi, per_w_lo + 1, per_w_lo)
        iota = lax.iota(jnp.int32, _L)
        rlo = iota
        rhi = _L + iota

        @pl.when(wid == _NUM_WORKERS - 1)
        def _():
            pltpu.sync_copy(tail_hbm, tl)
            pltpu.sync_copy(tl, out_hbm.at[pl.ds(n_full * _W * D, rem * D)])

        def load(i, b):
            blk = wid + i * _NUM_WORKERS
            pltpu.async_copy(
                tt_hbm.at[:, pl.ds(blk * _W, _W)], slab[b], gsem[b]
            )

        for b in range(2):
            @pl.when(b < nblk)
            def _():
                load(b, b)

        def body(i, carry):
            for b in range(2):
                it = i * 2 + b
                blk = wid + it * _NUM_WORKERS

                @pl.when(it < nblk)
                def _():
                    pltpu.make_async_copy(
                        tt_hbm.at[:, pl.ds(0, _W)], slab[b], gsem[b]
                    ).wait()

                    @pl.when(it >= 2)
                    def _():
                        pltpu.make_async_copy(
                            obuf[b], out_hbm.at[pl.ds(0, _W * D)], wsem[b]
                        ).wait()

                    def tbody(g, tc):
                        j = lax.shift_right_logical(g, 1)
                        rv = jnp.where(lax.bitwise_and(g, 1) == 0, rlo, rhi)
                        cv = jnp.broadcast_to(j, (_L,))
                        v = plsc.load_gather(slab[b], [rv, cv])
                        obuf[b][pl.ds(g * _L, _L)] = v
                        return tc

                    lax.fori_loop(0, _W * (D // _L), tbody, 0)

                    pltpu.async_copy(
                        obuf[b],
                        out_hbm.at[pl.ds(blk * _W * D, _W * D)],
                        wsem[b],
                    )

                    @pl.when(it + 2 < nblk)
                    def _():
                        load(it + 2, b)

            return carry

        n_outer = (per_w_lo + 1 + 1) // 2
        lax.fori_loop(0, n_outer, body, 0)

        for b in range(2):
            it_b = jnp.where((nblk - 1) % 2 == b, nblk - 1, nblk - 2)

            @pl.when(it_b >= 0)
            def _():
                pltpu.make_async_copy(
                    obuf[b], out_hbm.at[pl.ds(0, _W * D)], wsem[b]
                ).wait()

    return k(table_t, tail_flat)


@functools.partial(jax.jit, static_argnums=(2, 3, 4))
def _sc_gather_t(table, idx, B, H, D):
    C = B // _NUM_WORKERS
    mesh = plsc.VectorSubcoreMesh(core_axis_name="c", subcore_axis_name="s")

    @functools.partial(
        pl.kernel,
        mesh=mesh,
        out_type=jax.ShapeDtypeStruct((H, D, B), jnp.float32),
        scratch_types=[
            pltpu.VMEM((H, C), jnp.int32),
            pltpu.VMEM((C, D), jnp.float32),
            pltpu.VMEM((C, D), jnp.float32),
            pltpu.VMEM((D, C), jnp.float32),
            pltpu.VMEM((D, C), jnp.float32),
            pltpu.SemaphoreType.DMA,
            pltpu.SemaphoreType.DMA,
            pltpu.SemaphoreType.DMA,
            pltpu.SemaphoreType.DMA,
        ],
        compiler_params=pltpu.CompilerParams(
            use_tc_tiling_on_sc=False, needs_layout_passes=False
        ),
    )
    def k(table_hbm, idx_hbm, out_hbm, idxv, r0, r1, t0, t1, g0, g1, w0, w1):
        rows, trows = (r0, r1), (t0, t1)
        gsem, wsem = (g0, g1), (w0, w1)
        wid = lax.axis_index("s") * _NUM_CORES + lax.axis_index("c")
        base = wid * C
        iota = lax.iota(jnp.int32, _L)
        dvecs = [jnp.full((_L,), d, jnp.int32) for d in range(D)]

        for h in range(H):
            pltpu.async_copy(idx_hbm.at[pl.ds(h * B + base, C)], idxv.at[h], g0)
        for h in range(H):
            pltpu.make_async_copy(
                idx_hbm.at[pl.ds(h * B + base, C)], idxv.at[h], g0
            ).wait()
        for b in range(2):
            pltpu.async_copy(table_hbm.at[idxv.at[b]], rows[b], gsem[b])

        def outer(g, carry):
            for b in range(2):
                h = g * 2 + b
                pltpu.make_async_copy(
                    table_hbm.at[idxv.at[b]], rows[b], gsem[b]
                ).wait()

                @pl.when(g > 0)
                def _():
                    pltpu.make_async_copy(
                        trows[b], out_hbm.at[h, :, pl.ds(base, C)], wsem[b]
                    ).wait()

                def tbody(o, tc):
                    for u in range(_UNROLL):
                        jv = (o * _UNROLL + u) * _L + iota
                        vals = [
                            plsc.load_gather(rows[b], [jv, dvecs[d]])
                            for d in range(D)
                        ]
                        for d in range(D):
                            trows[b][d, pl.ds((o * _UNROLL + u) * _L, _L)] = (
                                vals[d]
                            )
                    return tc

                lax.fori_loop(0, C // (_L * _UNROLL), tbody, 0)

                pltpu.async_copy(
                    trows[b], out_hbm.at[h, :, pl.ds(base, C)], wsem[b]
                )

                @pl.when(h + 2 < H)
                def _():
                    pltpu.async_copy(
                        table_hbm.at[idxv.at[h + 2]], rows[b], gsem[b]
                    )

            return carry

        lax.fori_loop(0, H // 2, outer, 0)

        for b in range(2):
            pltpu.make_async_copy(
                trows[b], out_hbm.at[0, :, pl.ds(base, C)], wsem[b]
            ).wait()

    return k(table, idx)


def kernel(table, x):
    batch, hist = x.shape
    V, D = table.shape
    n_full = V // _W
    tail_flat = table[n_full * _W :, :].reshape(-1)
    t_flat = _sc_detile(table.T, tail_flat, V, D)
    t2 = t_flat.reshape(V, D)
    idx = x.T.reshape(-1).astype(jnp.int32)
    out_t = _sc_gather_t(t2, idx, batch, hist, D)
    return jnp.transpose(out_t, (2, 0, 1))


# speedup vs baseline: 1.0002x; 1.0002x over previous
import functools

import jax
import jax.numpy as jnp
from jax import lax
from jax.experimental import pallas as pl
from jax.experimental.pallas import tpu as pltpu
from jax.experimental.pallas import tpu_sc as plsc

_NUM_CORES = 2
_NUM_SUBCORES = 16
_NUM_WORKERS = _NUM_CORES * _NUM_SUBCORES
_L = 16
_W = 128
_UNROLL = 2


@functools.partial(jax.jit, static_argnums=(2, 3))
def _sc_detile(table_t, tail_flat, V, D):
    n_full = V // _W
    rem = V - n_full * _W
    per_w_lo = n_full // _NUM_WORKERS
    n_hi = n_full - per_w_lo * _NUM_WORKERS
    mesh = plsc.VectorSubcoreMesh(core_axis_name="c", subcore_axis_name="s")

    @functools.partial(
        pl.kernel,
        mesh=mesh,
        out_type=jax.ShapeDtypeStruct((V * D,), jnp.float32),
        scratch_types=[
            pltpu.VMEM((D, _W), jnp.float32),
            pltpu.VMEM((D, _W), jnp.float32),
            pltpu.VMEM((_W * D,), jnp.float32),
            pltpu.VMEM((_W * D,), jnp.float32),
            pltpu.VMEM((rem * D,), jnp.float32),
            pltpu.SemaphoreType.DMA,
            pltpu.SemaphoreType.DMA,
            pltpu.SemaphoreType.DMA,
            pltpu.SemaphoreType.DMA,
        ],
        compiler_params=pltpu.CompilerParams(
            use_tc_tiling_on_sc=True, needs_layout_passes=False
        ),
    )
    def k(tt_hbm, tail_hbm, out_hbm, s0, s1, o0, o1, tl, g0, g1, w0, w1):
        slab, obuf = (s0, s1), (o0, o1)
        gsem, wsem = (g0, g1), (w0, w1)
        wid = lax.axis_index("s") * _NUM_CORES + lax.axis_index("c")
        nblk = jnp.where(wid < n_hi, per_w_lo + 1, per_w_lo)
        iota = lax.iota(jnp.int32, _L)
        rlo = iota
        rhi = _L + iota

        @pl.when(wid == _NUM_WORKERS - 1)
        def _():
            pltpu.sync_copy(tail_hbm, tl)
            pltpu.sync_copy(tl, out_hbm.at[pl.ds(n_full * _W * D, rem * D)])

        def load(i, b):
            blk = wid + i * _NUM_WORKERS
            pltpu.async_copy(
                tt_hbm.at[:, pl.ds(blk * _W, _W)], slab[b], gsem[b]
            )

        for b in range(2):
            @pl.when(b < nblk)
            def _():
                load(b, b)

        def body(i, carry):
            for b in range(2):
                it = i * 2 + b
                blk = wid + it * _NUM_WORKERS

                @pl.when(it < nblk)
                def _():
                    pltpu.make_async_copy(
                        tt_hbm.at[:, pl.ds(0, _W)], slab[b], gsem[b]
                    ).wait()

                    @pl.when(it >= 2)
                    def _():
                        pltpu.make_async_copy(
                            obuf[b], out_hbm.at[pl.ds(0, _W * D)], wsem[b]
                        ).wait()

                    def tbody(o, tc):
                        for k in range(8):
                            rv = rlo if k % 2 == 0 else rhi
                            cv = jnp.broadcast_to(o * 4 + k // 2, (_L,))
                            v = plsc.load_gather(slab[b], [rv, cv])
                            obuf[b][pl.ds((o * 8 + k) * _L, _L)] = v
                        return tc

                    lax.fori_loop(0, _W * (D // _L) // 8, tbody, 0)

                    pltpu.async_copy(
                        obuf[b],
                        out_hbm.at[pl.ds(blk * _W * D, _W * D)],
                        wsem[b],
                    )

                    @pl.when(it + 2 < nblk)
                    def _():
                        load(it + 2, b)

            return carry

        n_outer = (per_w_lo + 1 + 1) // 2
        lax.fori_loop(0, n_outer, body, 0)

        for b in range(2):
            it_b = jnp.where((nblk - 1) % 2 == b, nblk - 1, nblk - 2)

            @pl.when(it_b >= 0)
            def _():
                pltpu.make_async_copy(
                    obuf[b], out_hbm.at[pl.ds(0, _W * D)], wsem[b]
                ).wait()

    return k(table_t, tail_flat)


@functools.partial(jax.jit, static_argnums=(2, 3, 4))
def _sc_gather_t(table, idx, B, H, D):
    C = B // _NUM_WORKERS
    mesh = plsc.VectorSubcoreMesh(core_axis_name="c", subcore_axis_name="s")

    @functools.partial(
        pl.kernel,
        mesh=mesh,
        out_type=jax.ShapeDtypeStruct((H, D, B), jnp.float32),
        scratch_types=[
            pltpu.VMEM((H, C), jnp.int32),
            pltpu.VMEM((C, D), jnp.float32),
            pltpu.VMEM((C, D), jnp.float32),
            pltpu.VMEM((D, C), jnp.float32),
            pltpu.VMEM((D, C), jnp.float32),
            pltpu.SemaphoreType.DMA,
            pltpu.SemaphoreType.DMA,
            pltpu.SemaphoreType.DMA,
            pltpu.SemaphoreType.DMA,
        ],
        compiler_params=pltpu.CompilerParams(
            use_tc_tiling_on_sc=False, needs_layout_passes=False
        ),
    )
    def k(table_hbm, idx_hbm, out_hbm, idxv, r0, r1, t0, t1, g0, g1, w0, w1):
        rows, trows = (r0, r1), (t0, t1)
        gsem, wsem = (g0, g1), (w0, w1)
        wid = lax.axis_index("s") * _NUM_CORES + lax.axis_index("c")
        base = wid * C
        iota = lax.iota(jnp.int32, _L)
        dvecs = [jnp.full((_L,), d, jnp.int32) for d in range(D)]

        for h in range(H):
            pltpu.async_copy(idx_hbm.at[pl.ds(h * B + base, C)], idxv.at[h], g0)
        for h in range(H):
            pltpu.make_async_copy(
                idx_hbm.at[pl.ds(h * B + base, C)], idxv.at[h], g0
            ).wait()
        for b in range(2):
            pltpu.async_copy(table_hbm.at[idxv.at[b]], rows[b], gsem[b])

        def outer(g, carry):
            for b in range(2):
                h = g * 2 + b
                pltpu.make_async_copy(
                    table_hbm.at[idxv.at[b]], rows[b], gsem[b]
                ).wait()

                @pl.when(g > 0)
                def _():
                    pltpu.make_async_copy(
                        trows[b], out_hbm.at[h, :, pl.ds(base, C)], wsem[b]
                    ).wait()

                def tbody(o, tc):
                    for u in range(_UNROLL):
                        jv = (o * _UNROLL + u) * _L + iota
                        vals = [
                            plsc.load_gather(rows[b], [jv, dvecs[d]])
                            for d in range(D)
                        ]
                        for d in range(D):
                            trows[b][d, pl.ds((o * _UNROLL + u) * _L, _L)] = (
                                vals[d]
                            )
                    return tc

                lax.fori_loop(0, C // (_L * _UNROLL), tbody, 0)

                pltpu.async_copy(
                    trows[b], out_hbm.at[h, :, pl.ds(base, C)], wsem[b]
                )

                @pl.when(h + 2 < H)
                def _():
                    pltpu.async_copy(
                        table_hbm.at[idxv.at[h + 2]], rows[b], gsem[b]
                    )

            return carry

        lax.fori_loop(0, H // 2, outer, 0)

        for b in range(2):
            pltpu.make_async_copy(
                trows[b], out_hbm.at[0, :, pl.ds(base, C)], wsem[b]
            ).wait()

    return k(table, idx)


def kernel(table, x):
    batch, hist = x.shape
    V, D = table.shape
    n_full = V // _W
    tail_flat = table[n_full * _W :, :].reshape(-1)
    t_flat = _sc_detile(table.T, tail_flat, V, D)
    t2 = t_flat.reshape(V, D)
    idx = x.T.reshape(-1).astype(jnp.int32)
    out_t = _sc_gather_t(t2, idx, batch, hist, D)
    return jnp.transpose(out_t, (2, 0, 1))


# speedup vs baseline: 1.1300x; 1.1297x over previous
import functools

import jax
import jax.numpy as jnp
from jax import lax
from jax.experimental import pallas as pl
from jax.experimental.pallas import tpu as pltpu
from jax.experimental.pallas import tpu_sc as plsc

_NUM_CORES = 2
_NUM_SUBCORES = 16
_NUM_WORKERS = _NUM_CORES * _NUM_SUBCORES
_L = 16
_W = 128
_UNROLL = 2


@functools.partial(jax.jit, static_argnums=(2, 3))
def _sc_detile(table_t, tail_flat, V, D):
    n_full = V // _W
    rem = V - n_full * _W
    per_w_lo = n_full // _NUM_WORKERS
    n_hi = n_full - per_w_lo * _NUM_WORKERS
    mesh = plsc.VectorSubcoreMesh(core_axis_name="c", subcore_axis_name="s")

    @functools.partial(
        pl.kernel,
        mesh=mesh,
        out_type=jax.ShapeDtypeStruct((V * D,), jnp.float32),
        scratch_types=[
            pltpu.VMEM((D, _W), jnp.float32),
            pltpu.VMEM((D, _W), jnp.float32),
            pltpu.VMEM((_W * D,), jnp.float32),
            pltpu.VMEM((_W * D,), jnp.float32),
            pltpu.VMEM((rem * D,), jnp.float32),
            pltpu.SemaphoreType.DMA,
            pltpu.SemaphoreType.DMA,
            pltpu.SemaphoreType.DMA,
            pltpu.SemaphoreType.DMA,
        ],
        compiler_params=pltpu.CompilerParams(
            use_tc_tiling_on_sc=True, needs_layout_passes=False
        ),
    )
    def k(tt_hbm, tail_hbm, out_hbm, s0, s1, o0, o1, tl, g0, g1, w0, w1):
        slab, obuf = (s0, s1), (o0, o1)
        gsem, wsem = (g0, g1), (w0, w1)
        wid = lax.axis_index("s") * _NUM_CORES + lax.axis_index("c")
        nblk = jnp.where(wid < n_hi, per_w_lo + 1, per_w_lo)
        iota = lax.iota(jnp.int32, _L)
        iota32 = iota * D

        @pl.when(wid == _NUM_WORKERS - 1)
        def _():
            pltpu.sync_copy(tail_hbm, tl)
            pltpu.sync_copy(tl, out_hbm.at[pl.ds(n_full * _W * D, rem * D)])

        def load(i, b):
            blk = wid + i * _NUM_WORKERS
            pltpu.async_copy(
                tt_hbm.at[:, pl.ds(blk * _W, _W)], slab[b], gsem[b]
            )

        for b in range(2):
            @pl.when(b < nblk)
            def _():
                load(b, b)

        def body(i, carry):
            for b in range(2):
                it = i * 2 + b
                blk = wid + it * _NUM_WORKERS

                @pl.when(it < nblk)
                def _():
                    pltpu.make_async_copy(
                        tt_hbm.at[:, pl.ds(0, _W)], slab[b], gsem[b]
                    ).wait()

                    @pl.when(it >= 2)
                    def _():
                        pltpu.make_async_copy(
                            obuf[b], out_hbm.at[pl.ds(0, _W * D)], wsem[b]
                        ).wait()

                    def tbody(o, tc):
                        for d in range(D):
                            v = slab[b][d, pl.ds(o * _L, _L)]
                            plsc.store_scatter(
                                obuf[b], [iota32 + (o * (_L * D) + d)], v
                            )
                        return tc

                    lax.fori_loop(0, _W // _L, tbody, 0)

                    pltpu.async_copy(
                        obuf[b],
                        out_hbm.at[pl.ds(blk * _W * D, _W * D)],
                        wsem[b],
                    )

                    @pl.when(it + 2 < nblk)
                    def _():
                        load(it + 2, b)

            return carry

        n_outer = (per_w_lo + 1 + 1) // 2
        lax.fori_loop(0, n_outer, body, 0)

        for b in range(2):
            it_b = jnp.where((nblk - 1) % 2 == b, nblk - 1, nblk - 2)

            @pl.when(it_b >= 0)
            def _():
                pltpu.make_async_copy(
                    obuf[b], out_hbm.at[pl.ds(0, _W * D)], wsem[b]
                ).wait()

    return k(table_t, tail_flat)


@functools.partial(jax.jit, static_argnums=(2, 3, 4))
def _sc_gather_t(table, idx, B, H, D):
    C = B // _NUM_WORKERS
    mesh = plsc.VectorSubcoreMesh(core_axis_name="c", subcore_axis_name="s")

    @functools.partial(
        pl.kernel,
        mesh=mesh,
        out_type=jax.ShapeDtypeStruct((H, D, B), jnp.float32),
        scratch_types=[
            pltpu.VMEM((H, C), jnp.int32),
            pltpu.VMEM((C, D), jnp.float32),
            pltpu.VMEM((C, D), jnp.float32),
            pltpu.VMEM((D, C), jnp.float32),
            pltpu.VMEM((D, C), jnp.float32),
            pltpu.SemaphoreType.DMA,
            pltpu.SemaphoreType.DMA,
            pltpu.SemaphoreType.DMA,
            pltpu.SemaphoreType.DMA,
        ],
        compiler_params=pltpu.CompilerParams(
            use_tc_tiling_on_sc=False, needs_layout_passes=False
        ),
    )
    def k(table_hbm, idx_hbm, out_hbm, idxv, r0, r1, t0, t1, g0, g1, w0, w1):
        rows, trows = (r0, r1), (t0, t1)
        gsem, wsem = (g0, g1), (w0, w1)
        wid = lax.axis_index("s") * _NUM_CORES + lax.axis_index("c")
        base = wid * C
        iota = lax.iota(jnp.int32, _L)
        dvecs = [jnp.full((_L,), d, jnp.int32) for d in range(D)]

        for h in range(H):
            pltpu.async_copy(idx_hbm.at[pl.ds(h * B + base, C)], idxv.at[h], g0)
        for h in range(H):
            pltpu.make_async_copy(
                idx_hbm.at[pl.ds(h * B + base, C)], idxv.at[h], g0
            ).wait()
        for b in range(2):
            pltpu.async_copy(table_hbm.at[idxv.at[b]], rows[b], gsem[b])

        def outer(g, carry):
            for b in range(2):
                h = g * 2 + b
                pltpu.make_async_copy(
                    table_hbm.at[idxv.at[b]], rows[b], gsem[b]
                ).wait()

                @pl.when(g > 0)
                def _():
                    pltpu.make_async_copy(
                        trows[b], out_hbm.at[h, :, pl.ds(base, C)], wsem[b]
                    ).wait()

                def tbody(o, tc):
                    for u in range(_UNROLL):
                        jv = (o * _UNROLL + u) * _L + iota
                        vals = [
                            plsc.load_gather(rows[b], [jv, dvecs[d]])
                            for d in range(D)
                        ]
                        for d in range(D):
                            trows[b][d, pl.ds((o * _UNROLL + u) * _L, _L)] = (
                                vals[d]
                            )
                    return tc

                lax.fori_loop(0, C // (_L * _UNROLL), tbody, 0)

                pltpu.async_copy(
                    trows[b], out_hbm.at[h, :, pl.ds(base, C)], wsem[b]
                )

                @pl.when(h + 2 < H)
                def _():
                    pltpu.async_copy(
                        table_hbm.at[idxv.at[h + 2]], rows[b], gsem[b]
                    )

            return carry

        lax.fori_loop(0, H // 2, outer, 0)

        for b in range(2):
            pltpu.make_async_copy(
                trows[b], out_hbm.at[0, :, pl.ds(base, C)], wsem[b]
            ).wait()

    return k(table, idx)


def kernel(table, x):
    batch, hist = x.shape
    V, D = table.shape
    n_full = V // _W
    tail_flat = table[n_full * _W :, :].reshape(-1)
    t_flat = _sc_detile(table.T, tail_flat, V, D)
    t2 = t_flat.reshape(V, D)
    idx = x.T.reshape(-1).astype(jnp.int32)
    out_t = _sc_gather_t(t2, idx, batch, hist, D)
    return jnp.transpose(out_t, (2, 0, 1))


# speedup vs baseline: 1.2796x; 1.1324x over previous
import functools

import jax
import jax.numpy as jnp
from jax import lax
from jax.experimental import pallas as pl
from jax.experimental.pallas import tpu as pltpu
from jax.experimental.pallas import tpu_sc as plsc

_NUM_CORES = 2
_NUM_SUBCORES = 16
_NUM_WORKERS = _NUM_CORES * _NUM_SUBCORES
_L = 16
_UNROLL = 4


@functools.partial(jax.jit, static_argnums=(2, 3, 4))
def _sc_gather_t(table, idx, B, H, D):
    C = B // _NUM_WORKERS
    mesh = plsc.VectorSubcoreMesh(core_axis_name="c", subcore_axis_name="s")

    @functools.partial(
        pl.kernel,
        mesh=mesh,
        out_type=jax.ShapeDtypeStruct((H, D, B), jnp.float32),
        scratch_types=[
            pltpu.VMEM((H, C), jnp.int32),
            pltpu.VMEM((C, D), jnp.float32),
            pltpu.VMEM((C, D), jnp.float32),
            pltpu.VMEM((D, C), jnp.float32),
            pltpu.VMEM((D, C), jnp.float32),
            pltpu.SemaphoreType.DMA,
            pltpu.SemaphoreType.DMA,
            pltpu.SemaphoreType.DMA,
            pltpu.SemaphoreType.DMA,
        ],
        compiler_params=pltpu.CompilerParams(
            use_tc_tiling_on_sc=False, needs_layout_passes=False
        ),
    )
    def k(table_hbm, idx_hbm, out_hbm, idxv, r0, r1, t0, t1, g0, g1, w0, w1):
        rows, trows = (r0, r1), (t0, t1)
        gsem, wsem = (g0, g1), (w0, w1)
        wid = lax.axis_index("s") * _NUM_CORES + lax.axis_index("c")
        base = wid * C
        iota = lax.iota(jnp.int32, _L)
        dvecs = [jnp.full((_L,), d, jnp.int32) for d in range(D)]

        for h in range(H):
            pltpu.async_copy(
                idx_hbm.at[pl.ds(h * B + base, C)], idxv.at[h], g0
            )
        for h in range(H):
            pltpu.make_async_copy(
                idx_hbm.at[pl.ds(h * B + base, C)], idxv.at[h], g0
            ).wait()
        for b in range(2):
            pltpu.async_copy(table_hbm.at[idxv.at[b]], rows[b], gsem[b])

        def outer(g, carry):
            for b in range(2):
                h = g * 2 + b
                pltpu.make_async_copy(
                    table_hbm.at[idxv.at[b]], rows[b], gsem[b]
                ).wait()

                @pl.when(g > 0)
                def _():
                    pltpu.make_async_copy(
                        trows[b], out_hbm.at[h, :, pl.ds(base, C)], wsem[b]
                    ).wait()

                def tbody(o, tc):
                    for u in range(_UNROLL):
                        jv = (o * _UNROLL + u) * _L + iota
                        vals = [
                            plsc.load_gather(rows[b], [jv, dvecs[d]])
                            for d in range(D)
                        ]
                        for d in range(D):
                            trows[b][d, pl.ds((o * _UNROLL + u) * _L, _L)] = (
                                vals[d]
                            )
                    return tc

                lax.fori_loop(0, C // (_L * _UNROLL), tbody, 0)

                pltpu.async_copy(
                    trows[b], out_hbm.at[h, :, pl.ds(base, C)], wsem[b]
                )

                @pl.when(h + 2 < H)
                def _():
                    pltpu.async_copy(
                        table_hbm.at[idxv.at[h + 2]], rows[b], gsem[b]
                    )

            return carry

        lax.fori_loop(0, H // 2, outer, 0)

        for b in range(2):
            pltpu.make_async_copy(
                trows[b], out_hbm.at[0, :, pl.ds(base, C)], wsem[b]
            ).wait()

    return k(table, idx)


def kernel(table, x):
    batch, hist = x.shape
    D = table.shape[1]
    idx = x.T.reshape(-1).astype(jnp.int32)
    out_t = _sc_gather_t(table, idx, batch, hist, D)
    return jnp.transpose(out_t, (2, 0, 1))
